# baseline (speedup 1.0000x reference)
import functools

import jax
import jax.numpy as jnp
from jax import lax
from jax.experimental import pallas as pl
from jax.experimental.pallas import tpu as pltpu
from jax.experimental.pallas import tpu_sc as plsc

_NSAMPLE = 1024
_NGROUP = 32
_SB = 256
_L = 16
_GR = 4


def _merge16(ak, av, bk, bv, need_hi=True):
    rbk = lax.rev(bk, (0,))
    rbv = lax.rev(bv, (0,))
    m = ak <= rbk
    lok = jnp.where(m, ak, rbk)
    lov = jnp.where(m, av, rbv)
    lok, lov = plsc.sort_key_val(lok, lov)
    hik = jnp.where(m, rbk, ak)
    hiv = jnp.where(m, rbv, av)
    if need_hi:
        hik, hiv = plsc.sort_key_val(hik, hiv)
    return lok, lov, hik, hiv


def _knn_gather_sc(xyzT, fea, B, N, D, S, K):
    info = plsc.get_sparse_core_info()
    NW = info.num_cores * info.num_subcores
    rows_total = B * S
    rpw = rows_total // NW
    nb = N // _L
    CAPL = 64
    inf = float("inf")
    n_grp = rpw // _GR
    GI = _GR * K

    mesh = plsc.VectorSubcoreMesh(core_axis_name="c", subcore_axis_name="s")

    def bf16r(x):
        u = lax.bitcast_convert_type(x, jnp.int32)
        u = (u + 0x7FFF + ((u >> 16) & 1)) & ~0xFFFF
        return lax.bitcast_convert_type(u, jnp.float32)

    @functools.partial(
        pl.kernel,
        mesh=mesh,
        compiler_params=pltpu.CompilerParams(needs_layout_passes=False, use_tc_tiling_on_sc=False),
        out_type=(
            jax.ShapeDtypeStruct((rows_total * K, D), jnp.float32),
            jax.ShapeDtypeStruct((rows_total * K * 4,), jnp.float32),
        ),
        scratch_types=[
            pltpu.VMEM((N,), jnp.float32),
            pltpu.VMEM((N,), jnp.float32),
            pltpu.VMEM((N,), jnp.float32),
            pltpu.VMEM((N + _L,), jnp.float32),
            pltpu.VMEM((N + _L,), jnp.float32),
            pltpu.VMEM((N + _L,), jnp.float32),
            pltpu.VMEM((N + _L,), jnp.float32),
            pltpu.VMEM((N + _L,), jnp.float32),
            pltpu.VMEM((N + _L,), jnp.float32),
            pltpu.VMEM((N + _L,), jnp.float32),
            pltpu.VMEM((N + _L,), jnp.float32),
            pltpu.VMEM((4 * _L * CAPL,), jnp.float32),
            pltpu.VMEM((4 * _L * CAPL,), jnp.int32),
            pltpu.VMEM((GI,), jnp.int32),
            pltpu.VMEM((GI,), jnp.int32),
            pltpu.VMEM((GI, D), jnp.float32),
            pltpu.VMEM((GI, D), jnp.float32),
            pltpu.VMEM((GI * 4,), jnp.float32),
            pltpu.VMEM((GI * 4,), jnp.float32),
            pltpu.SemaphoreType.DMA,
            pltpu.SemaphoreType.DMA,
            pltpu.SemaphoreType.DMA,
            pltpu.SemaphoreType.DMA,
            pltpu.SemaphoreType.DMA,
            pltpu.SemaphoreType.DMA,
        ],
    )
    def knn(xyzT_hbm, fea_hbm, gfea_hbm, rel_hbm,
            Xv, Yv, Zv, Xb, Yb, Zb, NXv, d2A, d2B, d2C, d2D, cd2, cidx,
            ib0, ib1, gb0, gb1, rb0, rb1,
            gs0, gs1, fs0, fs1, rs0, rs1):
        wid = lax.axis_index("s") * info.num_cores + lax.axis_index("c")
        w_per_b = NW // B
        bat = wid // w_per_b
        s0 = (wid % w_per_b) * rpw
        row0 = wid * rpw * K
        pltpu.sync_copy(xyzT_hbm.at[pl.ds(bat * 3 * N, N)], Xv)
        pltpu.sync_copy(xyzT_hbm.at[pl.ds(bat * 3 * N + N, N)], Yv)
        pltpu.sync_copy(xyzT_hbm.at[pl.ds(bat * 3 * N + 2 * N, N)], Zv)
        iota = lax.iota(jnp.int32, _L)
        ibufs = (ib0, ib1)
        gbufs = (gb0, gb1)
        rbufs = (rb0, rb1)
        gsems = (gs0, gs1)
        fsems = (fs0, fs1)
        rsems = (rs0, rs1)

        def prep(j, carry):
            x = Xv[pl.ds(j * _L, _L)]
            y = Yv[pl.ds(j * _L, _L)]
            z = Zv[pl.ds(j * _L, _L)]
            NXv[pl.ds(j * _L, _L)] = (x * x + y * y) + z * z
            Xb[pl.ds(j * _L, _L)] = bf16r(x)
            Yb[pl.ds(j * _L, _L)] = bf16r(y)
            Zb[pl.ds(j * _L, _L)] = bf16r(z)
            return carry
        lax.fori_loop(0, nb, prep, jnp.int32(0), unroll=4)

        zf = jnp.zeros((_L,), jnp.float32)
        for rb in rbufs:
            for j in range(GI * 4 // (4 * _L)):
                plsc.store_scatter(rb, [j * 4 * _L + iota * 4 + 3], zf)

        lane = lax.iota(jnp.int32, _L)
        lane15 = jnp.full((_L,), 15, jnp.int32)
        full_inf = jnp.full((_L,), inf, jnp.float32)
        zero_i = jnp.zeros((_L,), jnp.int32)

        def centroid(s):
            cx = jnp.full((_L,), Xv[pl.ds(s, _L)][0], jnp.float32)
            cy = jnp.full((_L,), Yv[pl.ds(s, _L)][0], jnp.float32)
            cz = jnp.full((_L,), Zv[pl.ds(s, _L)][0], jnp.float32)
            ns = cx * cx + cy * cy + cz * cz
            hx = jnp.full((_L,), Xb[pl.ds(s, _L)][0], jnp.float32)
            hy = jnp.full((_L,), Yb[pl.ds(s, _L)][0], jnp.float32)
            hz = jnp.full((_L,), Zb[pl.ds(s, _L)][0], jnp.float32)
            return (cx, cy, cz, ns, hx, hy, hz)

        def loads_at(j):
            return (Xb[pl.ds(j * _L, _L)], Yb[pl.ds(j * _L, _L)],
                    Zb[pl.ds(j * _L, _L)], NXv[pl.ds(j * _L, _L)])

        def scan_quad(cens):
            bufs = (d2A, d2B, d2C, d2D)

            def mk_body(store_min_slot):
                def p1(j, c):
                    xb, yb, zb, nx = c[:4]
                    mins = list(c[4:])
                    for q in range(4):
                        _, _, _, ns, hx, hy, hz = cens[q]
                        d = (ns + nx) - 2.0 * (hx * xb + hy * yb + hz * zb)
                        bufs[q][pl.ds(j * _L, _L)] = d
                        k = 2 * q + store_min_slot
                        mins[k] = jnp.minimum(mins[k], d)
                    return loads_at(j + 1) + tuple(mins)
                return p1

            c = (loads_at(0) + (full_inf,) * 8)
            c = lax.fori_loop(0, nb // 2, mk_body(0), c, unroll=16)
            c = lax.fori_loop(nb // 2, nb, mk_body(1), c, unroll=16)
            mins = c[4:]
            return tuple(jnp.maximum(mins[2 * q], mins[2 * q + 1])
                         for q in range(4))

        def bfly_max(v):
            for sh in (1, 2, 4, 8):
                v = jnp.maximum(v, v.at[lane ^ sh].get(
                    mode="promise_in_bounds"))
            return v

        lane_base = lane * CAPL

        def collect_quad(ts):
            bufs = (d2A, d2B, d2C, d2D)
            one = jnp.full((_L,), 1, jnp.int32)
            capm1 = jnp.full((_L,), CAPL - 1, jnp.int32)

            def d_at(j):
                return tuple(bufs[q][pl.ds(j * _L, _L)] for q in range(4))

            def p2(j, c):
                ds_ = c[:4]
                cur = c[4:]
                idxv = j * _L + iota
                out = []
                for q in range(4):
                    m = ds_[q] <= ts[q]
                    pos = (q * _L * CAPL) + lane_base + cur[q]
                    plsc.store_scatter(cd2, [pos], ds_[q], mask=m)
                    plsc.store_scatter(cidx, [pos], idxv, mask=m)
                    out.append(jnp.minimum(cur[q] + jnp.where(m, one, zero_i),
                                           capm1))
                return d_at(j + 1) + tuple(out)

            c = lax.fori_loop(0, nb, p2, d_at(0) + (zero_i,) * 4, unroll=16)
            return c[4:]

        def select_from(q, length):
            kmax = bfly_max(length)[0]
            base = (q * _L * CAPL) + lane_base

            def p3(k, c):
                ak, av, bk, bv = c
                pos = base + k
                ck = plsc.load_gather(cd2, [pos])
                cv = plsc.load_gather(cidx, [pos])
                ck = jnp.where(k < length, ck, full_inf)
                ck, cv = plsc.sort_key_val(ck, cv)
                ak, av, tk, tv = _merge16(ak, av, ck, cv, need_hi=True)
                bk, bv, _, _ = _merge16(bk, bv, tk, tv, need_hi=False)
                return (ak, av, bk, bv)

            sel = lax.fori_loop(
                0, kmax, p3, (full_inf, zero_i, full_inf, zero_i))
            return sel[1], sel[3]

        def emit_row(r2, buf, cen, av, bv):
            cx, cy, cz = cen[0], cen[1], cen[2]
            ibufs[buf][pl.ds(r2 * K, _L)] = av + bat * N
            ibufs[buf][pl.ds(r2 * K + _L, _L)] = bv + bat * N
            rb = rbufs[buf]
            for half, idxv in ((0, av), (1, bv)):
                gx = plsc.load_gather(Xv, [idxv]) - cx
                gy = plsc.load_gather(Yv, [idxv]) - cy
                gz = plsc.load_gather(Zv, [idxv]) - cz
                base = r2 * (4 * K) + half * (4 * _L)
                plsc.store_scatter(rb, [base + iota * 4 + 0], gx)
                plsc.store_scatter(rb, [base + iota * 4 + 1], gy)
                plsc.store_scatter(rb, [base + iota * 4 + 2], gz)

        def select_quad(gp, buf):
            r = gp * (2 * _GR) + buf * _GR
            s = s0 + r
            cens = [centroid(s + q) for q in range(4)]
            v2s = scan_quad(cens)
            ts = [bfly_max(v2s[q]) for q in range(4)]
            lens = collect_quad(ts)
            for q in range(4):
                av, bv = select_from(q, lens[q])
                emit_row(q, buf, cens[q], av, bv)
            return jnp.int32(0)

        def group(gp, carry):
            for buf in (0, 1):
                @pl.when(gp > 0)
                def _drain():
                    pltpu.make_async_copy(
                        fea_hbm.at[ibufs[buf]], gbufs[buf],
                        gsems[buf]).wait()
                    prev = row0 + ((gp - 1) * 2 + buf) * _GR * K
                    pltpu.make_async_copy(
                        gbufs[buf], gfea_hbm.at[pl.ds(prev, GI)],
                        fsems[buf]).start()
                    pltpu.make_async_copy(
                        rbufs[buf], rel_hbm.at[pl.ds(prev * 4, GI * 4)],
                        rsems[buf]).wait()

                select_quad(gp, buf)

                cur = row0 + (gp * 2 + buf) * _GR * K
                pltpu.make_async_copy(
                    rbufs[buf], rel_hbm.at[pl.ds(cur * 4, GI * 4)],
                    rsems[buf]).start()

                @pl.when(gp > 0)
                def _wait_feaout():
                    prev = row0 + ((gp - 1) * 2 + buf) * _GR * K
                    pltpu.make_async_copy(
                        gbufs[buf], gfea_hbm.at[pl.ds(prev, GI)],
                        fsems[buf]).wait()

                pltpu.make_async_copy(
                    fea_hbm.at[ibufs[buf]], gbufs[buf], gsems[buf]).start()
            return carry

        lax.fori_loop(0, n_grp // 2, group, jnp.int32(0))

        for buf in (0, 1):
            last = row0 + ((n_grp // 2 - 1) * 2 + buf) * _GR * K
            pltpu.make_async_copy(
                fea_hbm.at[ibufs[buf]], gbufs[buf], gsems[buf]).wait()
            pltpu.make_async_copy(
                gbufs[buf], gfea_hbm.at[pl.ds(last, GI)], fsems[buf]).start()
            pltpu.make_async_copy(
                rbufs[buf], rel_hbm.at[pl.ds(last * 4, GI * 4)],
                rsems[buf]).wait()
            pltpu.make_async_copy(
                gbufs[buf], gfea_hbm.at[pl.ds(last, GI)], fsems[buf]).wait()

    return knn(xyzT, fea)


def _mlp_block(rel_ref, fea_ref, w0a_ref, w0b_ref, b0_ref, w1_ref, b1_ref,
               w2_ref, b2_ref, out_ref, *, sb, k):
    bf = w0a_ref.dtype
    h = (jnp.dot(rel_ref[0].astype(bf), w0a_ref[...],
                 preferred_element_type=jnp.float32)
         + jnp.dot(fea_ref[0].astype(bf), w0b_ref[...],
                   preferred_element_type=jnp.float32)
         + b0_ref[...][None, :])
    h = jnp.maximum(h, 0.0).astype(w1_ref.dtype)
    h = jnp.maximum(jnp.dot(h, w1_ref[...], preferred_element_type=jnp.float32)
                    + b1_ref[...][None, :], 0.0).astype(w2_ref.dtype)
    h = jnp.maximum(jnp.dot(h, w2_ref[...], preferred_element_type=jnp.float32)
                    + b2_ref[...][None, :], 0.0)
    h = h.reshape(sb, k, h.shape[-1])
    out_ref[0] = jnp.max(h, axis=1)


def _mlp_maxpool(rel4, gfea, W0a, W0b, b0, W1, b1, W2, b2, *, sb, k):
    B, SK, _ = rel4.shape
    S = SK // k
    cout = W2.shape[1]
    grid = (B, S // sb)
    return pl.pallas_call(
        functools.partial(_mlp_block, sb=sb, k=k),
        grid=grid,
        in_specs=[
            pl.BlockSpec((1, sb * k, 4), lambda b, s: (b, s, 0)),
            pl.BlockSpec((1, sb * k, 64), lambda b, s: (b, s, 0)),
            pl.BlockSpec((4, W0a.shape[1]), lambda b, s: (0, 0)),
            pl.BlockSpec((64, W0b.shape[1]), lambda b, s: (0, 0)),
            pl.BlockSpec((b0.shape[0],), lambda b, s: (0,)),
            pl.BlockSpec((W1.shape[0], W1.shape[1]), lambda b, s: (0, 0)),
            pl.BlockSpec((b1.shape[0],), lambda b, s: (0,)),
            pl.BlockSpec((W2.shape[0], W2.shape[1]), lambda b, s: (0, 0)),
            pl.BlockSpec((b2.shape[0],), lambda b, s: (0,)),
        ],
        out_specs=pl.BlockSpec((1, sb, cout), lambda b, s: (b, s, 0)),
        out_shape=jax.ShapeDtypeStruct((B, S, cout), jnp.float32),
    )(rel4, gfea, W0a, W0b, b0, W1, b1, W2, b2)


def kernel(xyz, points_fea, W0, b0, W1, b1, W2, b2):
    B, N, _ = xyz.shape
    D = points_fea.shape[-1]
    S, K = _NSAMPLE, _NGROUP
    sampled_xyz = xyz[:, :S, :]
    xyzT = jnp.transpose(xyz, (0, 2, 1)).reshape(-1)
    fea2d = points_fea.reshape(B * N, D)
    gfea, rel = _knn_gather_sc(xyzT, fea2d, B, N, D, S, K)
    bf = jnp.bfloat16
    rel4 = rel.reshape(B, S * K, 4)
    gfea = gfea.reshape(B, S * K, D)
    W0a = jnp.concatenate([W0[:3], jnp.zeros((1, W0.shape[1]), W0.dtype)], 0)
    out_fea = _mlp_maxpool(rel4, gfea, W0a.astype(bf), W0[3:].astype(bf), b0,
                           W1.astype(bf), b1, W2.astype(bf), b2, sb=_SB, k=K)
    return (sampled_xyz, out_fea)


# speedup vs baseline: 1.0640x; 1.0640x over previous
import functools

import jax
import jax.numpy as jnp
from jax import lax
from jax.experimental import pallas as pl
from jax.experimental.pallas import tpu as pltpu
from jax.experimental.pallas import tpu_sc as plsc

_NSAMPLE = 1024
_NGROUP = 32
_SB = 256
_L = 16
_GR = 4


def _merge16(ak, av, bk, bv, need_hi=True):
    rbk = lax.rev(bk, (0,))
    rbv = lax.rev(bv, (0,))
    m = ak <= rbk
    lok = jnp.where(m, ak, rbk)
    lov = jnp.where(m, av, rbv)
    lok, lov = plsc.sort_key_val(lok, lov)
    hik = jnp.where(m, rbk, ak)
    hiv = jnp.where(m, rbv, av)
    if need_hi:
        hik, hiv = plsc.sort_key_val(hik, hiv)
    return lok, lov, hik, hiv


def _knn_gather_sc(xyzT, fea, B, N, D, S, K):
    info = plsc.get_sparse_core_info()
    NW = info.num_cores * info.num_subcores
    rows_total = B * S
    rpw = rows_total // NW
    nb = N // _L
    CAPL = 64
    inf = float("inf")
    n_grp = rpw // _GR
    GI = _GR * K

    mesh = plsc.VectorSubcoreMesh(core_axis_name="c", subcore_axis_name="s")

    def bf16r(x):
        u = lax.bitcast_convert_type(x, jnp.int32)
        u = (u + 0x7FFF + ((u >> 16) & 1)) & ~0xFFFF
        return lax.bitcast_convert_type(u, jnp.float32)

    @functools.partial(
        pl.kernel,
        mesh=mesh,
        compiler_params=pltpu.CompilerParams(needs_layout_passes=False, use_tc_tiling_on_sc=False),
        out_type=(
            jax.ShapeDtypeStruct((rows_total * K, D), jnp.float32),
            jax.ShapeDtypeStruct((rows_total * K * 4,), jnp.float32),
        ),
        scratch_types=[
            pltpu.VMEM((N,), jnp.float32),
            pltpu.VMEM((N,), jnp.float32),
            pltpu.VMEM((N,), jnp.float32),
            pltpu.VMEM((N + _L,), jnp.float32),
            pltpu.VMEM((N + _L,), jnp.float32),
            pltpu.VMEM((N + _L,), jnp.float32),
            pltpu.VMEM((N + _L,), jnp.float32),
            pltpu.VMEM((N + _L,), jnp.float32),
            pltpu.VMEM((N + _L,), jnp.float32),
            pltpu.VMEM((N + _L,), jnp.float32),
            pltpu.VMEM((N + _L,), jnp.float32),
            pltpu.VMEM((4 * _L * CAPL,), jnp.float32),
            pltpu.VMEM((4 * _L * CAPL,), jnp.int32),
            pltpu.VMEM((GI,), jnp.int32),
            pltpu.VMEM((GI,), jnp.int32),
            pltpu.VMEM((GI, D), jnp.float32),
            pltpu.VMEM((GI, D), jnp.float32),
            pltpu.VMEM((GI * 4,), jnp.float32),
            pltpu.VMEM((GI * 4,), jnp.float32),
            pltpu.SemaphoreType.DMA,
            pltpu.SemaphoreType.DMA,
            pltpu.SemaphoreType.DMA,
            pltpu.SemaphoreType.DMA,
            pltpu.SemaphoreType.DMA,
            pltpu.SemaphoreType.DMA,
        ],
    )
    def knn(xyzT_hbm, fea_hbm, gfea_hbm, rel_hbm,
            Xv, Yv, Zv, Xb, Yb, Zb, NXv, d2A, d2B, d2C, d2D, cd2, cidx,
            ib0, ib1, gb0, gb1, rb0, rb1,
            gs0, gs1, fs0, fs1, rs0, rs1):
        wid = lax.axis_index("s") * info.num_cores + lax.axis_index("c")
        w_per_b = NW // B
        bat = wid // w_per_b
        s0 = (wid % w_per_b) * rpw
        row0 = wid * rpw * K
        pltpu.sync_copy(xyzT_hbm.at[pl.ds(bat * 3 * N, N)], Xv)
        pltpu.sync_copy(xyzT_hbm.at[pl.ds(bat * 3 * N + N, N)], Yv)
        pltpu.sync_copy(xyzT_hbm.at[pl.ds(bat * 3 * N + 2 * N, N)], Zv)
        iota = lax.iota(jnp.int32, _L)
        ibufs = (ib0, ib1)
        gbufs = (gb0, gb1)
        rbufs = (rb0, rb1)
        gsems = (gs0, gs1)
        fsems = (fs0, fs1)
        rsems = (rs0, rs1)

        def prep(j, carry):
            x = Xv[pl.ds(j * _L, _L)]
            y = Yv[pl.ds(j * _L, _L)]
            z = Zv[pl.ds(j * _L, _L)]
            NXv[pl.ds(j * _L, _L)] = (x * x + y * y) + z * z
            Xb[pl.ds(j * _L, _L)] = bf16r(x)
            Yb[pl.ds(j * _L, _L)] = bf16r(y)
            Zb[pl.ds(j * _L, _L)] = bf16r(z)
            return carry
        lax.fori_loop(0, nb, prep, jnp.int32(0), unroll=4)

        zf = jnp.zeros((_L,), jnp.float32)
        for rb in rbufs:
            for j in range(GI * 4 // (4 * _L)):
                plsc.store_scatter(rb, [j * 4 * _L + iota * 4 + 3], zf)

        lane = lax.iota(jnp.int32, _L)
        lane15 = jnp.full((_L,), 15, jnp.int32)
        full_inf = jnp.full((_L,), inf, jnp.float32)
        zero_i = jnp.zeros((_L,), jnp.int32)

        def centroid(s):
            cx = jnp.full((_L,), Xv[pl.ds(s, _L)][0], jnp.float32)
            cy = jnp.full((_L,), Yv[pl.ds(s, _L)][0], jnp.float32)
            cz = jnp.full((_L,), Zv[pl.ds(s, _L)][0], jnp.float32)
            ns = cx * cx + cy * cy + cz * cz
            hx = jnp.full((_L,), Xb[pl.ds(s, _L)][0], jnp.float32)
            hy = jnp.full((_L,), Yb[pl.ds(s, _L)][0], jnp.float32)
            hz = jnp.full((_L,), Zb[pl.ds(s, _L)][0], jnp.float32)
            return (cx, cy, cz, ns, hx, hy, hz)

        def loads_at(j):
            return (Xb[pl.ds(j * _L, _L)], Yb[pl.ds(j * _L, _L)],
                    Zb[pl.ds(j * _L, _L)], NXv[pl.ds(j * _L, _L)])

        def scan_quad(cens):
            bufs = (d2A, d2B, d2C, d2D)

            def mk_body(store_min_slot):
                def p1(j, c):
                    xb, yb, zb, nx = c[:4]
                    mins = list(c[4:])
                    for q in range(4):
                        _, _, _, ns, hx, hy, hz = cens[q]
                        d = (ns + nx) - 2.0 * (hx * xb + hy * yb + hz * zb)
                        bufs[q][pl.ds(j * _L, _L)] = d
                        k = 2 * q + store_min_slot
                        mins[k] = jnp.minimum(mins[k], d)
                    return loads_at(j + 1) + tuple(mins)
                return p1

            c = (loads_at(0) + (full_inf,) * 8)
            c = lax.fori_loop(0, nb // 2, mk_body(0), c, unroll=8)
            c = lax.fori_loop(nb // 2, nb, mk_body(1), c, unroll=8)
            mins = c[4:]
            return tuple(jnp.maximum(mins[2 * q], mins[2 * q + 1])
                         for q in range(4))

        def bfly_max(v):
            for sh in (1, 2, 4, 8):
                v = jnp.maximum(v, v.at[lane ^ sh].get(
                    mode="promise_in_bounds"))
            return v

        lane_base = lane * CAPL

        def collect_quad(ts):
            bufs = (d2A, d2B, d2C, d2D)
            one = jnp.full((_L,), 1, jnp.int32)
            capm1 = jnp.full((_L,), CAPL - 1, jnp.int32)

            def d_at(j):
                return tuple(bufs[q][pl.ds(j * _L, _L)] for q in range(4))

            def p2(j, c):
                ds_ = c[:4]
                cur = c[4:]
                idxv = j * _L + iota
                out = []
                for q in range(4):
                    m = ds_[q] <= ts[q]
                    pos = (q * _L * CAPL) + lane_base + cur[q]
                    plsc.store_scatter(cd2, [pos], ds_[q], mask=m)
                    plsc.store_scatter(cidx, [pos], idxv, mask=m)
                    out.append(jnp.minimum(cur[q] + jnp.where(m, one, zero_i),
                                           capm1))
                return d_at(j + 1) + tuple(out)

            c = lax.fori_loop(0, nb, p2, d_at(0) + (zero_i,) * 4, unroll=8)
            return c[4:]

        def select_from(q, length):
            kmax = bfly_max(length)[0]
            base = (q * _L * CAPL) + lane_base

            def p3(k, c):
                ak, av, bk, bv = c
                pos = base + k
                ck = plsc.load_gather(cd2, [pos])
                cv = plsc.load_gather(cidx, [pos])
                ck = jnp.where(k < length, ck, full_inf)
                ck, cv = plsc.sort_key_val(ck, cv)
                ak, av, tk, tv = _merge16(ak, av, ck, cv, need_hi=True)
                bk, bv, _, _ = _merge16(bk, bv, tk, tv, need_hi=False)
                return (ak, av, bk, bv)

            sel = lax.fori_loop(
                0, kmax, p3, (full_inf, zero_i, full_inf, zero_i))
            return sel[1], sel[3]

        def emit_row(r2, buf, cen, av, bv):
            cx, cy, cz = cen[0], cen[1], cen[2]
            ibufs[buf][pl.ds(r2 * K, _L)] = av + bat * N
            ibufs[buf][pl.ds(r2 * K + _L, _L)] = bv + bat * N
            rb = rbufs[buf]
            for half, idxv in ((0, av), (1, bv)):
                gx = plsc.load_gather(Xv, [idxv]) - cx
                gy = plsc.load_gather(Yv, [idxv]) - cy
                gz = plsc.load_gather(Zv, [idxv]) - cz
                base = r2 * (4 * K) + half * (4 * _L)
                plsc.store_scatter(rb, [base + iota * 4 + 0], gx)
                plsc.store_scatter(rb, [base + iota * 4 + 1], gy)
                plsc.store_scatter(rb, [base + iota * 4 + 2], gz)

        def select_quad(gp, buf):
            r = gp * (2 * _GR) + buf * _GR
            s = s0 + r
            cens = [centroid(s + q) for q in range(4)]
            v2s = scan_quad(cens)
            ts = [bfly_max(v2s[q]) for q in range(4)]
            lens = collect_quad(ts)
            for q in range(4):
                av, bv = select_from(q, lens[q])
                emit_row(q, buf, cens[q], av, bv)
            return jnp.int32(0)

        def group(gp, carry):
            for buf in (0, 1):
                @pl.when(gp > 0)
                def _drain():
                    pltpu.make_async_copy(
                        fea_hbm.at[ibufs[buf]], gbufs[buf],
                        gsems[buf]).wait()
                    prev = row0 + ((gp - 1) * 2 + buf) * _GR * K
                    pltpu.make_async_copy(
                        gbufs[buf], gfea_hbm.at[pl.ds(prev, GI)],
                        fsems[buf]).start()
                    pltpu.make_async_copy(
                        rbufs[buf], rel_hbm.at[pl.ds(prev * 4, GI * 4)],
                        rsems[buf]).wait()

                select_quad(gp, buf)

                cur = row0 + (gp * 2 + buf) * _GR * K
                pltpu.make_async_copy(
                    rbufs[buf], rel_hbm.at[pl.ds(cur * 4, GI * 4)],
                    rsems[buf]).start()

                @pl.when(gp > 0)
                def _wait_feaout():
                    prev = row0 + ((gp - 1) * 2 + buf) * _GR * K
                    pltpu.make_async_copy(
                        gbufs[buf], gfea_hbm.at[pl.ds(prev, GI)],
                        fsems[buf]).wait()

                pltpu.make_async_copy(
                    fea_hbm.at[ibufs[buf]], gbufs[buf], gsems[buf]).start()
            return carry

        lax.fori_loop(0, n_grp // 2, group, jnp.int32(0))

        for buf in (0, 1):
            last = row0 + ((n_grp // 2 - 1) * 2 + buf) * _GR * K
            pltpu.make_async_copy(
                fea_hbm.at[ibufs[buf]], gbufs[buf], gsems[buf]).wait()
            pltpu.make_async_copy(
                gbufs[buf], gfea_hbm.at[pl.ds(last, GI)], fsems[buf]).start()
            pltpu.make_async_copy(
                rbufs[buf], rel_hbm.at[pl.ds(last * 4, GI * 4)],
                rsems[buf]).wait()
            pltpu.make_async_copy(
                gbufs[buf], gfea_hbm.at[pl.ds(last, GI)], fsems[buf]).wait()

    return knn(xyzT, fea)


def _mlp_block(rel_ref, fea_ref, w0a_ref, w0b_ref, b0_ref, w1_ref, b1_ref,
               w2_ref, b2_ref, out_ref, *, sb, k):
    h = (jnp.dot(rel_ref[0], w0a_ref[...], preferred_element_type=jnp.float32)
         + jnp.dot(fea_ref[0], w0b_ref[...], preferred_element_type=jnp.float32)
         + b0_ref[...][None, :])
    h = jnp.maximum(h, 0.0).astype(w1_ref.dtype)
    h = jnp.maximum(jnp.dot(h, w1_ref[...], preferred_element_type=jnp.float32)
                    + b1_ref[...][None, :], 0.0).astype(w2_ref.dtype)
    h = jnp.maximum(jnp.dot(h, w2_ref[...], preferred_element_type=jnp.float32)
                    + b2_ref[...][None, :], 0.0)
    h = h.reshape(sb, k, h.shape[-1])
    out_ref[0] = jnp.max(h, axis=1)


def _mlp_maxpool(rel4, gfea, W0a, W0b, b0, W1, b1, W2, b2, *, sb, k):
    B, SK, _ = rel4.shape
    S = SK // k
    cout = W2.shape[1]
    grid = (B, S // sb)
    return pl.pallas_call(
        functools.partial(_mlp_block, sb=sb, k=k),
        grid=grid,
        in_specs=[
            pl.BlockSpec((1, sb * k, 4), lambda b, s: (b, s, 0)),
            pl.BlockSpec((1, sb * k, 64), lambda b, s: (b, s, 0)),
            pl.BlockSpec((4, W0a.shape[1]), lambda b, s: (0, 0)),
            pl.BlockSpec((64, W0b.shape[1]), lambda b, s: (0, 0)),
            pl.BlockSpec((b0.shape[0],), lambda b, s: (0,)),
            pl.BlockSpec((W1.shape[0], W1.shape[1]), lambda b, s: (0, 0)),
            pl.BlockSpec((b1.shape[0],), lambda b, s: (0,)),
            pl.BlockSpec((W2.shape[0], W2.shape[1]), lambda b, s: (0, 0)),
            pl.BlockSpec((b2.shape[0],), lambda b, s: (0,)),
        ],
        out_specs=pl.BlockSpec((1, sb, cout), lambda b, s: (b, s, 0)),
        out_shape=jax.ShapeDtypeStruct((B, S, cout), jnp.float32),
    )(rel4, gfea, W0a, W0b, b0, W1, b1, W2, b2)


def kernel(xyz, points_fea, W0, b0, W1, b1, W2, b2):
    B, N, _ = xyz.shape
    D = points_fea.shape[-1]
    S, K = _NSAMPLE, _NGROUP
    sampled_xyz = xyz[:, :S, :]
    xyzT = jnp.transpose(xyz, (0, 2, 1)).reshape(-1)
    fea2d = points_fea.reshape(B * N, D)
    gfea, rel = _knn_gather_sc(xyzT, fea2d, B, N, D, S, K)
    bf = jnp.bfloat16
    rel4 = rel.reshape(B, S * K, 4).astype(bf)
    gfea = gfea.reshape(B, S * K, D).astype(bf)
    W0a = jnp.concatenate([W0[:3], jnp.zeros((1, W0.shape[1]), W0.dtype)], 0)
    out_fea = _mlp_maxpool(rel4, gfea, W0a.astype(bf), W0[3:].astype(bf), b0,
                           W1.astype(bf), b1, W2.astype(bf), b2, sb=_SB, k=K)
    return (sampled_xyz, out_fea)
